# baseline (speedup 1.0000x reference)
import functools

import jax
import jax.numpy as jnp
from jax import lax
from jax.experimental import pallas as pl
from jax.experimental.pallas import tpu as pltpu
from jax.experimental.pallas import tpu_sc as plsc

N = 16384
F = 128
NC, NS = 2, 16
NW = NC * NS
TPW = N // NW
CH = 128
NCH = TPW // CH


def _sc_gather(tok, head_emb, e1t, e2t, e3t):
    mesh = plsc.VectorSubcoreMesh(
        core_axis_name="c", subcore_axis_name="s", num_cores=NC, num_subcores=NS
    )

    @functools.partial(
        pl.kernel,
        compiler_params=pltpu.CompilerParams(use_tc_tiling_on_sc=False),
        out_type=(
            jax.ShapeDtypeStruct((N, 128), jnp.float32),
            jax.ShapeDtypeStruct((32, N), jnp.float32),
            jax.ShapeDtypeStruct((16, N), jnp.float32),
        ),
        mesh=mesh,
        scratch_types=[
            pltpu.VMEM((TPW,), jnp.int32),
            pltpu.VMEM((NCH, CH), jnp.int32),
            pltpu.VMEM((NCH, CH), jnp.int32),
            pltpu.VMEM((NCH, CH), jnp.int32),
            pltpu.VMEM((NCH, CH), jnp.int32),
            pltpu.VMEM((TPW, 128), jnp.float32),
            pltpu.VMEM((32, TPW), jnp.float32),
            pltpu.VMEM((16, TPW), jnp.float32),
            pltpu.SemaphoreType.DMA,
        ],
    )
    def k(tok_hbm, he_hbm, e1t_hbm, e2t_hbm, e3t_hbm,
          gh_hbm, g1t_hbm, g23t_hbm,
          tok_v, ih_v, i1_v, i2_v, i3_v, gh_v, g1t_v, g23t_v, sem):
        wid = lax.axis_index("s") * NC + lax.axis_index("c")
        base = wid * TPW
        pltpu.sync_copy(tok_hbm.at[pl.ds(base, TPW)], tok_v)
        zeros = jnp.zeros((16,), jnp.float32)
        for j in range(TPW // 16):
            v = tok_v[pl.ds(j * 16, 16)]
            r, c = divmod(j * 16, CH)
            s = pl.ds(c, 16)
            ih_v[r, s] = jnp.clip(v, 0, 9999)
            i1_v[r, s] = jnp.clip(v - 10000, 0, 49999)
            i2_v[r, s] = jnp.clip(v - 60000, 0, 129999)
            i3_v[r, s] = jnp.clip(v - 190000, 0, 809999)
            for z in range(10, 16):
                g23t_v[z, pl.ds(j * 16, 16)] = zeros
        ABL = 1
        cps = []
        for j in range(NCH):
            cols = pl.ds(j * CH, CH)
            if ABL != 3:
                cps.append(pltpu.make_async_copy(
                    he_hbm.at[ih_v.at[j]], gh_v.at[cols], sem))
            if ABL in (0, 2):
                for kk in range(32):
                    cps.append(pltpu.make_async_copy(
                        e1t_hbm.at[kk].at[i1_v.at[j]], g1t_v.at[kk, cols], sem))
            if ABL == 0:
                for kk in range(8):
                    cps.append(pltpu.make_async_copy(
                        e2t_hbm.at[kk].at[i2_v.at[j]], g23t_v.at[kk, cols], sem))
                for kk in range(2):
                    cps.append(pltpu.make_async_copy(
                        e3t_hbm.at[kk].at[i3_v.at[j]], g23t_v.at[8 + kk, cols], sem))
        for cp in cps:
            cp.start()
        for cp in cps:
            cp.wait()
        toks = pl.ds(base, TPW)
        pltpu.sync_copy(gh_v, gh_hbm.at[toks])
        pltpu.sync_copy(g1t_v, g1t_hbm.at[:, toks])
        pltpu.sync_copy(g23t_v, g23t_hbm.at[:, toks])

    return k(tok, head_emb, e1t, e2t, e3t)


BM = 512


def _tc_body(tokr_r, gh_r, g1t_r, g23t_r, wh_r, w1_r, w23_r, out_r):
    tr = tokr_r[...][0:1, :]
    m0r = (tr < 10000).astype(jnp.float32)
    m0full = lax.dot_general(m0r, jnp.ones((1, 128), jnp.float32),
                             (((0,), (0,)), ((), ())),
                             preferred_element_type=jnp.float32)
    acc = jnp.dot(gh_r[...], wh_r[...], preferred_element_type=jnp.float32) * m0full
    m1 = (tr >= 10000) & (tr < 60000)
    g1t = jnp.where(m1, g1t_r[...], 0.0)
    acc += lax.dot_general(g1t, w1_r[...], (((0,), (0,)), ((), ())),
                           preferred_element_type=jnp.float32)
    m2 = (tr >= 60000) & (tr < 190000)
    m3 = tr >= 190000
    row = lax.broadcasted_iota(jnp.int32, (16, BM), 0)
    r8 = row < 8
    m23 = (r8 & m2) | (~r8 & (row < 10) & m3)
    g23t = jnp.where(m23, g23t_r[...], 0.0)
    acc += lax.dot_general(g23t, w23_r[...], (((0,), (0,)), ((), ())),
                           preferred_element_type=jnp.float32)
    out_r[...] = acc


def _tc_project(tokrow, gh, g1t, g23t, head_W, W1, W23):
    grid = (N // BM,)
    return pl.pallas_call(
        _tc_body,
        grid=grid,
        in_specs=[
            pl.BlockSpec((8, BM), lambda i: (0, i)),
            pl.BlockSpec((BM, 128), lambda i: (i, 0)),
            pl.BlockSpec((32, BM), lambda i: (0, i)),
            pl.BlockSpec((16, BM), lambda i: (0, i)),
            pl.BlockSpec((128, 128), lambda i: (0, 0)),
            pl.BlockSpec((32, 128), lambda i: (0, 0)),
            pl.BlockSpec((16, 128), lambda i: (0, 0)),
        ],
        out_specs=pl.BlockSpec((BM, 128), lambda i: (i, 0)),
        out_shape=jax.ShapeDtypeStruct((N, F), jnp.float32),
    )(tokrow, gh, g1t, g23t, head_W, W1, W23)


def kernel(input, head_emb, head_W, emb1, W1, emb2, W2, emb3, W3):
    gh, g1t, g23t = _sc_gather(input, head_emb, emb1.T, emb2.T, emb3.T)
    W23 = jnp.concatenate([W2, W3, jnp.zeros((6, 128), jnp.float32)], axis=0)
    tokrow = jnp.broadcast_to(input[None, :], (8, N))
    return _tc_project(tokrow, gh, g1t, g23t, head_W, W1, W23)


# speedup vs baseline: 2.6990x; 2.6990x over previous
import functools

import jax
import jax.numpy as jnp
from jax import lax
from jax.experimental import pallas as pl
from jax.experimental.pallas import tpu as pltpu
from jax.experimental.pallas import tpu_sc as plsc

N = 16384
F = 128
NC, NS = 2, 16
NW = NC * NS
TPW = N // NW
CH = 128
NCH = TPW // CH


def _sc_gather(tok, head_emb, e1t, e2t, e3t):
    mesh = plsc.VectorSubcoreMesh(
        core_axis_name="c", subcore_axis_name="s", num_cores=NC, num_subcores=NS
    )

    @functools.partial(
        pl.kernel,
        compiler_params=pltpu.CompilerParams(use_tc_tiling_on_sc=False),
        out_type=(
            jax.ShapeDtypeStruct((N, 128), jnp.float32),
            jax.ShapeDtypeStruct((32, N), jnp.float32),
            jax.ShapeDtypeStruct((16, N), jnp.float32),
        ),
        mesh=mesh,
        scratch_types=[
            pltpu.VMEM((TPW,), jnp.int32),
            pltpu.VMEM((NCH, CH), jnp.int32),
            pltpu.VMEM((NCH, CH), jnp.int32),
            pltpu.VMEM((NCH, CH), jnp.int32),
            pltpu.VMEM((NCH, CH), jnp.int32),
            pltpu.VMEM((CH, 128), jnp.float32),
            pltpu.VMEM((32, TPW), jnp.float32),
            pltpu.VMEM((16, TPW), jnp.float32),
            pltpu.VMEM_SHARED((10000, 128), jnp.float32),
            pltpu.SemaphoreType.DMA,
            pltpu.SemaphoreType.DMA,
            pltpu.SemaphoreType.DMA,
        ],
    )
    def k(tok_hbm, he_hbm, e1t_hbm, e2t_hbm, e3t_hbm,
          gh_hbm, g1t_hbm, g23t_hbm,
          tok_v, ih_v, i1_v, i2_v, i3_v, gh_v, g1t_v, g23t_v, he_sh, sem, semh, semw):
        sid = lax.axis_index("s")
        wid = sid * NC + lax.axis_index("c")
        base = wid * TPW
        @pl.when(sid == 0)
        def _load_head():
            pltpu.sync_copy(he_hbm, he_sh)
        pltpu.sync_copy(tok_hbm.at[pl.ds(base, TPW)], tok_v)
        zeros = jnp.zeros((16,), jnp.float32)
        for j in range(TPW // 16):
            v = tok_v[pl.ds(j * 16, 16)]
            r, c = divmod(j * 16, CH)
            s = pl.ds(c, 16)
            ih_v[r, s] = jnp.clip(v, 0, 9999)
            i1_v[r, s] = jnp.clip(v - 10000, 0, 49999)
            i2_v[r, s] = jnp.clip(v - 60000, 0, 129999)
            i3_v[r, s] = jnp.clip(v - 190000, 0, 809999)
            for z in range(10, 16):
                g23t_v[z, pl.ds(j * 16, 16)] = zeros
        cps = []
        for j in range(NCH):
            cols = pl.ds(j * CH, CH)
            for kk in range(32):
                cps.append(pltpu.make_async_copy(
                    e1t_hbm.at[kk].at[i1_v.at[j]], g1t_v.at[kk, cols], sem))
            for kk in range(8):
                cps.append(pltpu.make_async_copy(
                    e2t_hbm.at[kk].at[i2_v.at[j]], g23t_v.at[kk, cols], sem))
            for kk in range(2):
                cps.append(pltpu.make_async_copy(
                    e3t_hbm.at[kk].at[i3_v.at[j]], g23t_v.at[8 + kk, cols], sem))
        for cp in cps:
            cp.start()
        plsc.subcore_barrier()
        for j in range(NCH):
            hc = pltpu.make_async_copy(he_sh.at[ih_v.at[j]], gh_v, semh)
            hc.start()
            hc.wait()
            wr = pltpu.make_async_copy(
                gh_v, gh_hbm.at[pl.ds(base + j * CH, CH)], semw)
            wr.start()
            wr.wait()
        for cp in cps:
            cp.wait()
        toks = pl.ds(base, TPW)
        pltpu.sync_copy(g1t_v, g1t_hbm.at[:, toks])
        pltpu.sync_copy(g23t_v, g23t_hbm.at[:, toks])

    return k(tok, head_emb, e1t, e2t, e3t)


BM = 512


def _tc_body(tokr_r, gh_r, g1t_r, g23t_r, wh_r, w1_r, w23_r, out_r):
    tr = tokr_r[...][0:1, :]
    m0r = (tr < 10000).astype(jnp.float32)
    m0full = lax.dot_general(m0r, jnp.ones((1, 128), jnp.float32),
                             (((0,), (0,)), ((), ())),
                             preferred_element_type=jnp.float32)
    acc = jnp.dot(gh_r[...], wh_r[...], preferred_element_type=jnp.float32) * m0full
    m1 = (tr >= 10000) & (tr < 60000)
    g1t = jnp.where(m1, g1t_r[...], 0.0)
    acc += lax.dot_general(g1t, w1_r[...], (((0,), (0,)), ((), ())),
                           preferred_element_type=jnp.float32)
    m2 = (tr >= 60000) & (tr < 190000)
    m3 = tr >= 190000
    row = lax.broadcasted_iota(jnp.int32, (16, BM), 0)
    r8 = row < 8
    m23 = (r8 & m2) | (~r8 & (row < 10) & m3)
    g23t = jnp.where(m23, g23t_r[...], 0.0)
    acc += lax.dot_general(g23t, w23_r[...], (((0,), (0,)), ((), ())),
                           preferred_element_type=jnp.float32)
    out_r[...] = acc


def _tc_project(tokrow, gh, g1t, g23t, head_W, W1, W23):
    grid = (N // BM,)
    return pl.pallas_call(
        _tc_body,
        grid=grid,
        in_specs=[
            pl.BlockSpec((8, BM), lambda i: (0, i)),
            pl.BlockSpec((BM, 128), lambda i: (i, 0)),
            pl.BlockSpec((32, BM), lambda i: (0, i)),
            pl.BlockSpec((16, BM), lambda i: (0, i)),
            pl.BlockSpec((128, 128), lambda i: (0, 0)),
            pl.BlockSpec((32, 128), lambda i: (0, 0)),
            pl.BlockSpec((16, 128), lambda i: (0, 0)),
        ],
        out_specs=pl.BlockSpec((BM, 128), lambda i: (i, 0)),
        out_shape=jax.ShapeDtypeStruct((N, F), jnp.float32),
    )(tokrow, gh, g1t, g23t, head_W, W1, W23)


def kernel(input, head_emb, head_W, emb1, W1, emb2, W2, emb3, W3):
    gh, g1t, g23t = _sc_gather(input, head_emb, emb1.T, emb2.T, emb3.T)
    W23 = jnp.concatenate([W2, W3, jnp.zeros((6, 128), jnp.float32)], axis=0)
    tokrow = jnp.broadcast_to(input[None, :], (8, N))
    return _tc_project(tokrow, gh, g1t, g23t, head_W, W1, W23)


# speedup vs baseline: 4.9686x; 1.8409x over previous
import functools

import jax
import jax.numpy as jnp
from jax import lax
from jax.experimental import pallas as pl
from jax.experimental.pallas import tpu as pltpu
from jax.experimental.pallas import tpu_sc as plsc

N = 16384
F = 128
NC, NS = 2, 16
NW = NC * NS
TPW = N // NW
CH = 128
NCH = TPW // CH


def _sc_head(tok, head_emb):
    mesh = plsc.VectorSubcoreMesh(
        core_axis_name="c", subcore_axis_name="s", num_cores=NC, num_subcores=NS
    )

    @functools.partial(
        pl.kernel,
        compiler_params=pltpu.CompilerParams(use_tc_tiling_on_sc=False),
        out_type=jax.ShapeDtypeStruct((N, 128), jnp.float32),
        mesh=mesh,
        scratch_types=[
            pltpu.VMEM((TPW,), jnp.int32),
            pltpu.VMEM((NCH, CH), jnp.int32),
            pltpu.VMEM((CH, 128), jnp.float32),
            pltpu.VMEM_SHARED((10000, 128), jnp.float32),
            pltpu.SemaphoreType.DMA,
            pltpu.SemaphoreType.DMA,
        ],
    )
    def k(tok_hbm, he_hbm, gh_hbm, tok_v, ih_v, gh_v, he_sh, semh, semw):
        sid = lax.axis_index("s")
        wid = sid * NC + lax.axis_index("c")
        base = wid * TPW
        pltpu.sync_copy(he_hbm.at[pl.ds(sid * 625, 625)],
                        he_sh.at[pl.ds(sid * 625, 625)])
        pltpu.sync_copy(tok_hbm.at[pl.ds(base, TPW)], tok_v)
        for j in range(TPW // 16):
            v = tok_v[pl.ds(j * 16, 16)]
            r, c = divmod(j * 16, CH)
            ih_v[r, pl.ds(c, 16)] = jnp.clip(v, 0, 9999)
        plsc.subcore_barrier()
        for j in range(NCH):
            hc = pltpu.make_async_copy(he_sh.at[ih_v.at[j]], gh_v, semh)
            hc.start()
            hc.wait()
            wr = pltpu.make_async_copy(
                gh_v, gh_hbm.at[pl.ds(base + j * CH, CH)], semw)
            wr.start()
            wr.wait()

    return k(tok, head_emb)


def _sc_tails(tok, e1t, e2t, e3t):
    mesh = plsc.VectorSubcoreMesh(
        core_axis_name="c", subcore_axis_name="s", num_cores=NC, num_subcores=NS
    )

    @functools.partial(
        pl.kernel,
        compiler_params=pltpu.CompilerParams(use_tc_tiling_on_sc=False),
        out_type=(
            jax.ShapeDtypeStruct((32, N), jnp.float32),
            jax.ShapeDtypeStruct((16, N), jnp.float32),
        ),
        mesh=mesh,
        scratch_types=[
            pltpu.VMEM((TPW,), jnp.int32),
            pltpu.VMEM((NCH, 16, CH), jnp.int32),
            pltpu.VMEM((NCH, 4, CH), jnp.int32),
            pltpu.VMEM((NCH, CH), jnp.int32),
            pltpu.VMEM((32, CH), jnp.float32),
            pltpu.VMEM((16, TPW), jnp.float32),
            pltpu.VMEM_SHARED((2, 810000), jnp.float32),
            pltpu.SemaphoreType.DMA,
            pltpu.SemaphoreType.DMA,
        ],
    )
    def k(tok_hbm, e1t_hbm, e2t_hbm, e3t_hbm, g1t_hbm, g23t_hbm,
          tok_v, i1k_v, i2k_v, i3_v, g1t_v, g23t_v, t_sh, sem, semw):
        sid = lax.axis_index("s")
        wid = sid * NC + lax.axis_index("c")
        base = wid * TPW
        pltpu.sync_copy(tok_hbm.at[pl.ds(base, TPW)], tok_v)
        zeros = jnp.zeros((16,), jnp.float32)
        for j in range(TPW // 16):
            v = tok_v[pl.ds(j * 16, 16)]
            r, c = divmod(j * 16, CH)
            s = pl.ds(c, 16)
            v1 = jnp.clip(v - 10000, 0, 49999)
            for m in range(16):
                i1k_v[r, m, s] = v1 + (m * 50000)
            v2 = jnp.clip(v - 60000, 0, 129999)
            for m in range(4):
                i2k_v[r, m, s] = v2 + (m * 130000)
            i3_v[r, s] = jnp.clip(v - 190000, 0, 809999)
            for z in range(10, 16):
                g23t_v[z, pl.ds(j * 16, 16)] = zeros
        pltpu.sync_copy(e1t_hbm.at[pl.ds(sid, 1)],
                        t_sh.at[pl.ds(0, 1), pl.ds(sid * 50000, 50000)])
        pltpu.sync_copy(e1t_hbm.at[pl.ds(sid + 16, 1)],
                        t_sh.at[pl.ds(1, 1), pl.ds(sid * 50000, 50000)])
        plsc.subcore_barrier()
        for j in range(NCH):
            cps = [pltpu.make_async_copy(
                t_sh.at[kk // 16].at[i1k_v.at[j, kk % 16]], g1t_v.at[kk], sem)
                for kk in range(32)]
            for cp in cps:
                cp.start()
            for cp in cps:
                cp.wait()
            wr = pltpu.make_async_copy(
                g1t_v, g1t_hbm.at[:, pl.ds(base + j * CH, CH)], semw)
            wr.start()
            wr.wait()
        plsc.subcore_barrier()
        @pl.when(sid < 8)
        def _load_e2():
            pltpu.sync_copy(
                e2t_hbm.at[pl.ds(sid, 1)],
                t_sh.at[pl.ds(sid // 4, 1), pl.ds((sid % 4) * 130000, 130000)])
        plsc.subcore_barrier()
        cps = []
        for j in range(NCH):
            cols = pl.ds(j * CH, CH)
            for kk in range(8):
                cps.append(pltpu.make_async_copy(
                    t_sh.at[kk // 4].at[i2k_v.at[j, kk % 4]],
                    g23t_v.at[kk, cols], sem))
        for cp in cps:
            cp.start()
        for cp in cps:
            cp.wait()
        plsc.subcore_barrier()
        @pl.when(sid < 4)
        def _load_e3():
            half = pl.ds((sid % 2) * 405000, 405000)
            pltpu.sync_copy(e3t_hbm.at[pl.ds(sid // 2, 1), half],
                            t_sh.at[pl.ds(sid // 2, 1), half])
        plsc.subcore_barrier()
        cps = []
        for j in range(NCH):
            cols = pl.ds(j * CH, CH)
            for kk in range(2):
                cps.append(pltpu.make_async_copy(
                    t_sh.at[kk].at[i3_v.at[j]], g23t_v.at[8 + kk, cols], sem))
        for cp in cps:
            cp.start()
        for cp in cps:
            cp.wait()
        pltpu.sync_copy(g23t_v, g23t_hbm.at[:, pl.ds(base, TPW)])

    return k(tok, e1t, e2t, e3t)


BM = 512


def _tc_body(tokr_r, gh_r, g1t_r, g23t_r, wh_r, w1_r, w23_r, out_r):
    tr = tokr_r[...][0:1, :]
    m0r = (tr < 10000).astype(jnp.float32)
    m0full = lax.dot_general(m0r, jnp.ones((1, 128), jnp.float32),
                             (((0,), (0,)), ((), ())),
                             preferred_element_type=jnp.float32)
    acc = jnp.dot(gh_r[...], wh_r[...], preferred_element_type=jnp.float32) * m0full
    m1 = (tr >= 10000) & (tr < 60000)
    g1t = jnp.where(m1, g1t_r[...], 0.0)
    acc += lax.dot_general(g1t, w1_r[...], (((0,), (0,)), ((), ())),
                           preferred_element_type=jnp.float32)
    m2 = (tr >= 60000) & (tr < 190000)
    m3 = tr >= 190000
    row = lax.broadcasted_iota(jnp.int32, (16, BM), 0)
    r8 = row < 8
    m23 = (r8 & m2) | (~r8 & (row < 10) & m3)
    g23t = jnp.where(m23, g23t_r[...], 0.0)
    acc += lax.dot_general(g23t, w23_r[...], (((0,), (0,)), ((), ())),
                           preferred_element_type=jnp.float32)
    out_r[...] = acc


def _tc_project(tokrow, gh, g1t, g23t, head_W, W1, W23):
    grid = (N // BM,)
    return pl.pallas_call(
        _tc_body,
        grid=grid,
        in_specs=[
            pl.BlockSpec((8, BM), lambda i: (0, i)),
            pl.BlockSpec((BM, 128), lambda i: (i, 0)),
            pl.BlockSpec((32, BM), lambda i: (0, i)),
            pl.BlockSpec((16, BM), lambda i: (0, i)),
            pl.BlockSpec((128, 128), lambda i: (0, 0)),
            pl.BlockSpec((32, 128), lambda i: (0, 0)),
            pl.BlockSpec((16, 128), lambda i: (0, 0)),
        ],
        out_specs=pl.BlockSpec((BM, 128), lambda i: (i, 0)),
        out_shape=jax.ShapeDtypeStruct((N, F), jnp.float32),
    )(tokrow, gh, g1t, g23t, head_W, W1, W23)


def kernel(input, head_emb, head_W, emb1, W1, emb2, W2, emb3, W3):
    gh = _sc_head(input, head_emb)
    g1t, g23t = _sc_tails(input, emb1.T, emb2.T, emb3.T)
    W23 = jnp.concatenate([W2, W3, jnp.zeros((6, 128), jnp.float32)], axis=0)
    tokrow = jnp.broadcast_to(input[None, :], (8, N))
    return _tc_project(tokrow, gh, g1t, g23t, head_W, W1, W23)


# speedup vs baseline: 5.1179x; 1.0300x over previous
import functools

import jax
import jax.numpy as jnp
from jax import lax
from jax.experimental import pallas as pl
from jax.experimental.pallas import tpu as pltpu
from jax.experimental.pallas import tpu_sc as plsc

N = 16384
F = 128
NC, NS = 2, 16
NW = NC * NS
TPW = N // NW
CH = 128
NCH = TPW // CH


def _sc_head(tok, head_emb):
    mesh = plsc.VectorSubcoreMesh(
        core_axis_name="c", subcore_axis_name="s", num_cores=NC, num_subcores=NS
    )

    @functools.partial(
        pl.kernel,
        compiler_params=pltpu.CompilerParams(use_tc_tiling_on_sc=False),
        out_type=jax.ShapeDtypeStruct((N, 128), jnp.float32),
        mesh=mesh,
        scratch_types=[
            pltpu.VMEM((TPW,), jnp.int32),
            pltpu.VMEM((NCH, CH), jnp.int32),
            pltpu.VMEM((2, CH, 128), jnp.float32),
            pltpu.VMEM_SHARED((10000, 128), jnp.float32),
            pltpu.SemaphoreType.DMA,
            pltpu.SemaphoreType.DMA,
            pltpu.SemaphoreType.DMA,
            pltpu.SemaphoreType.DMA,
        ],
    )
    def k(tok_hbm, he_hbm, gh_hbm, tok_v, ih_v, gh_v, he_sh,
          semh0, semh1, semw0, semw1):
        sid = lax.axis_index("s")
        wid = sid * NC + lax.axis_index("c")
        base = wid * TPW
        pltpu.sync_copy(he_hbm.at[pl.ds(sid * 625, 625)],
                        he_sh.at[pl.ds(sid * 625, 625)])
        pltpu.sync_copy(tok_hbm.at[pl.ds(base, TPW)], tok_v)
        for j in range(TPW // 16):
            v = tok_v[pl.ds(j * 16, 16)]
            r, c = divmod(j * 16, CH)
            ih_v[r, pl.ds(c, 16)] = jnp.clip(v, 0, 9999)
        plsc.subcore_barrier()
        semh = [semh0, semh1]
        semw = [semw0, semw1]
        gc = [pltpu.make_async_copy(he_sh.at[ih_v.at[j]], gh_v.at[j % 2],
                                    semh[j % 2]) for j in range(NCH)]
        wc = [pltpu.make_async_copy(gh_v.at[j % 2],
                                    gh_hbm.at[pl.ds(base + j * CH, CH)],
                                    semw[j % 2]) for j in range(NCH)]
        gc[0].start()
        for j in range(NCH):
            gc[j].wait()
            wc[j].start()
            if j + 1 < NCH:
                if j >= 1:
                    wc[j - 1].wait()
                gc[j + 1].start()
        wc[NCH - 1].wait()

    return k(tok, head_emb)


def _sc_tails(tok, e1t, e2t, e3t):
    mesh = plsc.VectorSubcoreMesh(
        core_axis_name="c", subcore_axis_name="s", num_cores=NC, num_subcores=NS
    )

    @functools.partial(
        pl.kernel,
        compiler_params=pltpu.CompilerParams(use_tc_tiling_on_sc=False),
        out_type=(
            jax.ShapeDtypeStruct((32, N), jnp.float32),
            jax.ShapeDtypeStruct((16, N), jnp.float32),
        ),
        mesh=mesh,
        scratch_types=[
            pltpu.VMEM((TPW,), jnp.int32),
            pltpu.VMEM((NCH, 16, CH), jnp.int32),
            pltpu.VMEM((NCH, 4, CH), jnp.int32),
            pltpu.VMEM((NCH, CH), jnp.int32),
            pltpu.VMEM((2, 32, CH), jnp.float32),
            pltpu.VMEM((16, TPW), jnp.float32),
            pltpu.VMEM_SHARED((2, 810000), jnp.float32),
            pltpu.SemaphoreType.DMA,
            pltpu.SemaphoreType.DMA,
            pltpu.SemaphoreType.DMA,
            pltpu.SemaphoreType.DMA,
            pltpu.SemaphoreType.DMA,
        ],
    )
    def k(tok_hbm, e1t_hbm, e2t_hbm, e3t_hbm, g1t_hbm, g23t_hbm,
          tok_v, i1k_v, i2k_v, i3_v, g1t_v, g23t_v, t_sh,
          sem, semst, semg1, semw0, semw1):
        sid = lax.axis_index("s")
        wid = sid * NC + lax.axis_index("c")
        base = wid * TPW
        st1 = pltpu.make_async_copy(
            e1t_hbm.at[pl.ds(sid, 1)],
            t_sh.at[pl.ds(0, 1), pl.ds(sid * 50000, 50000)], semst)
        st2 = pltpu.make_async_copy(
            e1t_hbm.at[pl.ds(sid + 16, 1)],
            t_sh.at[pl.ds(1, 1), pl.ds(sid * 50000, 50000)], semst)
        st1.start()
        st2.start()
        pltpu.sync_copy(tok_hbm.at[pl.ds(base, TPW)], tok_v)
        zeros = jnp.zeros((16,), jnp.float32)
        for j in range(TPW // 16):
            v = tok_v[pl.ds(j * 16, 16)]
            r, c = divmod(j * 16, CH)
            s = pl.ds(c, 16)
            v1 = jnp.clip(v - 10000, 0, 49999)
            for m in range(16):
                i1k_v[r, m, s] = v1 + (m * 50000)
            v2 = jnp.clip(v - 60000, 0, 129999)
            for m in range(4):
                i2k_v[r, m, s] = v2 + (m * 130000)
            i3_v[r, s] = jnp.clip(v - 190000, 0, 809999)
            for z in range(10, 16):
                g23t_v[z, pl.ds(j * 16, 16)] = zeros
        st1.wait()
        st2.wait()
        plsc.subcore_barrier()
        semg = [sem, semg1]
        semw = [semw0, semw1]
        gc = [[pltpu.make_async_copy(
            t_sh.at[kk // 16].at[i1k_v.at[j, kk % 16]],
            g1t_v.at[j % 2, kk], semg[j % 2]) for kk in range(32)]
            for j in range(NCH)]
        wc = [pltpu.make_async_copy(
            g1t_v.at[j % 2], g1t_hbm.at[:, pl.ds(base + j * CH, CH)],
            semw[j % 2]) for j in range(NCH)]
        for cp in gc[0]:
            cp.start()
        for j in range(NCH):
            for cp in gc[j]:
                cp.wait()
            wc[j].start()
            if j + 1 < NCH:
                if j >= 1:
                    wc[j - 1].wait()
                for cp in gc[j + 1]:
                    cp.start()
        wc[NCH - 1].wait()
        plsc.subcore_barrier()
        @pl.when(sid < 8)
        def _load_e2():
            pltpu.sync_copy(
                e2t_hbm.at[pl.ds(sid, 1)],
                t_sh.at[pl.ds(sid // 4, 1), pl.ds((sid % 4) * 130000, 130000)])
        plsc.subcore_barrier()
        cps = []
        for j in range(NCH):
            cols = pl.ds(j * CH, CH)
            for kk in range(8):
                cps.append(pltpu.make_async_copy(
                    t_sh.at[kk // 4].at[i2k_v.at[j, kk % 4]],
                    g23t_v.at[kk, cols], sem))
        for cp in cps:
            cp.start()
        for cp in cps:
            cp.wait()
        plsc.subcore_barrier()
        @pl.when(sid < 4)
        def _load_e3():
            half = pl.ds((sid % 2) * 405000, 405000)
            pltpu.sync_copy(e3t_hbm.at[pl.ds(sid // 2, 1), half],
                            t_sh.at[pl.ds(sid // 2, 1), half])
        plsc.subcore_barrier()
        cps = []
        for j in range(NCH):
            cols = pl.ds(j * CH, CH)
            for kk in range(2):
                cps.append(pltpu.make_async_copy(
                    t_sh.at[kk].at[i3_v.at[j]], g23t_v.at[8 + kk, cols], sem))
        for cp in cps:
            cp.start()
        for cp in cps:
            cp.wait()
        pltpu.sync_copy(g23t_v, g23t_hbm.at[:, pl.ds(base, TPW)])

    return k(tok, e1t, e2t, e3t)


BM = 512


def _tc_body(tokr_r, gh_r, g1t_r, g23t_r, wh_r, w1_r, w23_r, out_r):
    tr = tokr_r[...][0:1, :]
    m0r = (tr < 10000).astype(jnp.float32)
    m0full = lax.dot_general(m0r, jnp.ones((1, 128), jnp.float32),
                             (((0,), (0,)), ((), ())),
                             preferred_element_type=jnp.float32)
    acc = jnp.dot(gh_r[...], wh_r[...], preferred_element_type=jnp.float32) * m0full
    m1 = (tr >= 10000) & (tr < 60000)
    g1t = jnp.where(m1, g1t_r[...], 0.0)
    acc += lax.dot_general(g1t, w1_r[...], (((0,), (0,)), ((), ())),
                           preferred_element_type=jnp.float32)
    m2 = (tr >= 60000) & (tr < 190000)
    m3 = tr >= 190000
    row = lax.broadcasted_iota(jnp.int32, (16, BM), 0)
    r8 = row < 8
    m23 = (r8 & m2) | (~r8 & (row < 10) & m3)
    g23t = jnp.where(m23, g23t_r[...], 0.0)
    acc += lax.dot_general(g23t, w23_r[...], (((0,), (0,)), ((), ())),
                           preferred_element_type=jnp.float32)
    out_r[...] = acc


def _tc_project(tokrow, gh, g1t, g23t, head_W, W1, W23):
    grid = (N // BM,)
    return pl.pallas_call(
        _tc_body,
        grid=grid,
        in_specs=[
            pl.BlockSpec((8, BM), lambda i: (0, i)),
            pl.BlockSpec((BM, 128), lambda i: (i, 0)),
            pl.BlockSpec((32, BM), lambda i: (0, i)),
            pl.BlockSpec((16, BM), lambda i: (0, i)),
            pl.BlockSpec((128, 128), lambda i: (0, 0)),
            pl.BlockSpec((32, 128), lambda i: (0, 0)),
            pl.BlockSpec((16, 128), lambda i: (0, 0)),
        ],
        out_specs=pl.BlockSpec((BM, 128), lambda i: (i, 0)),
        out_shape=jax.ShapeDtypeStruct((N, F), jnp.float32),
    )(tokrow, gh, g1t, g23t, head_W, W1, W23)


def kernel(input, head_emb, head_W, emb1, W1, emb2, W2, emb3, W3):
    gh = _sc_head(input, head_emb)
    g1t, g23t = _sc_tails(input, emb1.T, emb2.T, emb3.T)
    W23 = jnp.concatenate([W2, W3, jnp.zeros((6, 128), jnp.float32)], axis=0)
    tokrow = jnp.broadcast_to(input[None, :], (8, N))
    return _tc_project(tokrow, gh, g1t, g23t, head_W, W1, W23)


# speedup vs baseline: 5.3706x; 1.0494x over previous
import functools

import jax
import jax.numpy as jnp
from jax import lax
from jax.experimental import pallas as pl
from jax.experimental.pallas import tpu as pltpu
from jax.experimental.pallas import tpu_sc as plsc

N = 16384
F = 128
NC, NS = 2, 16
NW = NC * NS
TPW = N // NW
CH = 128
NCH = TPW // CH


def _sc_head(tok, head_emb):
    mesh = plsc.VectorSubcoreMesh(
        core_axis_name="c", subcore_axis_name="s", num_cores=NC, num_subcores=NS
    )

    @functools.partial(
        pl.kernel,
        compiler_params=pltpu.CompilerParams(use_tc_tiling_on_sc=False),
        out_type=jax.ShapeDtypeStruct((N, 128), jnp.float32),
        mesh=mesh,
        scratch_types=[
            pltpu.VMEM((TPW,), jnp.int32),
            pltpu.VMEM((NCH, CH), jnp.int32),
            pltpu.VMEM((2, CH, 128), jnp.float32),
            pltpu.VMEM_SHARED((10000, 128), jnp.float32),
            pltpu.SemaphoreType.DMA,
            pltpu.SemaphoreType.DMA,
            pltpu.SemaphoreType.DMA,
            pltpu.SemaphoreType.DMA,
        ],
    )
    def k(tok_hbm, he_hbm, gh_hbm, tok_v, ih_v, gh_v, he_sh,
          semh0, semh1, semw0, semw1):
        sid = lax.axis_index("s")
        wid = sid * NC + lax.axis_index("c")
        base = wid * TPW
        pltpu.sync_copy(he_hbm.at[pl.ds(sid * 625, 625)],
                        he_sh.at[pl.ds(sid * 625, 625)])
        pltpu.sync_copy(tok_hbm.at[pl.ds(base, TPW)], tok_v)
        for j in range(TPW // 16):
            v = tok_v[pl.ds(j * 16, 16)]
            r, c = divmod(j * 16, CH)
            ih_v[r, pl.ds(c, 16)] = jnp.clip(v, 0, 9999)
        plsc.subcore_barrier()
        semh = [semh0, semh1]
        semw = [semw0, semw1]
        gc = [pltpu.make_async_copy(he_sh.at[ih_v.at[j]], gh_v.at[j % 2],
                                    semh[j % 2]) for j in range(NCH)]
        wc = [pltpu.make_async_copy(gh_v.at[j % 2],
                                    gh_hbm.at[pl.ds(base + j * CH, CH)],
                                    semw[j % 2]) for j in range(NCH)]
        gc[0].start()
        for j in range(NCH):
            gc[j].wait()
            wc[j].start()
            if j + 1 < NCH:
                if j >= 1:
                    wc[j - 1].wait()
                gc[j + 1].start()
        wc[NCH - 1].wait()

    return k(tok, head_emb)


def _sc_tails(tok, e1t, e2t, e3t):
    mesh = plsc.VectorSubcoreMesh(
        core_axis_name="c", subcore_axis_name="s", num_cores=NC, num_subcores=NS
    )

    @functools.partial(
        pl.kernel,
        compiler_params=pltpu.CompilerParams(use_tc_tiling_on_sc=False),
        out_type=(
            jax.ShapeDtypeStruct((32, N), jnp.float32),
            jax.ShapeDtypeStruct((16, N), jnp.float32),
        ),
        mesh=mesh,
        scratch_types=[
            pltpu.VMEM((TPW,), jnp.int32),
            pltpu.VMEM((NCH, 16, CH), jnp.int32),
            pltpu.VMEM((NCH, 4, CH), jnp.int32),
            pltpu.VMEM((NCH, CH), jnp.int32),
            pltpu.VMEM((2, 32, CH), jnp.float32),
            pltpu.VMEM((16, TPW), jnp.float32),
            pltpu.VMEM_SHARED((2, 810000), jnp.float32),
            pltpu.SemaphoreType.DMA,
            pltpu.SemaphoreType.DMA,
            pltpu.SemaphoreType.DMA,
            pltpu.SemaphoreType.DMA,
            pltpu.SemaphoreType.DMA,
        ],
    )
    def k(tok_hbm, e1t_hbm, e2t_hbm, e3t_hbm, g1t_hbm, g23t_hbm,
          tok_v, i1k_v, i2k_v, i3_v, g1t_v, g23t_v, t_sh,
          sem, semst, semg1, semw0, semw1):
        sid = lax.axis_index("s")
        wid = sid * NC + lax.axis_index("c")
        base = wid * TPW
        st1 = pltpu.make_async_copy(
            e1t_hbm.at[pl.ds(sid, 1)],
            t_sh.at[pl.ds(0, 1), pl.ds(sid * 50000, 50000)], semst)
        st2 = pltpu.make_async_copy(
            e1t_hbm.at[pl.ds(sid + 16, 1)],
            t_sh.at[pl.ds(1, 1), pl.ds(sid * 50000, 50000)], semst)
        st1.start()
        st2.start()
        pltpu.sync_copy(tok_hbm.at[pl.ds(base, TPW)], tok_v)
        zeros = jnp.zeros((16,), jnp.float32)
        for j in range(TPW // 16):
            v = tok_v[pl.ds(j * 16, 16)]
            r, c = divmod(j * 16, CH)
            s = pl.ds(c, 16)
            v1 = jnp.clip(v - 10000, 0, 49999)
            for m in range(16):
                i1k_v[r, m, s] = v1 + (m * 50000)
            v2 = jnp.clip(v - 60000, 0, 129999)
            for m in range(4):
                i2k_v[r, m, s] = v2 + (m * 130000)
            i3_v[r, s] = jnp.clip(v - 190000, 0, 809999)
            for z in range(10, 16):
                g23t_v[z, pl.ds(j * 16, 16)] = zeros
        st1.wait()
        st2.wait()
        plsc.subcore_barrier()
        semg = [sem, semg1]
        semw = [semw0, semw1]
        gc = [[pltpu.make_async_copy(
            t_sh.at[kk // 16].at[i1k_v.at[j, kk % 16]],
            g1t_v.at[j % 2, kk], semg[j % 2]) for kk in range(32)]
            for j in range(NCH)]
        wc = [pltpu.make_async_copy(
            g1t_v.at[j % 2], g1t_hbm.at[:, pl.ds(base + j * CH, CH)],
            semw[j % 2]) for j in range(NCH)]
        for cp in gc[0]:
            cp.start()
        for j in range(NCH):
            for cp in gc[j]:
                cp.wait()
            wc[j].start()
            if j + 1 < NCH:
                if j >= 1:
                    wc[j - 1].wait()
                for cp in gc[j + 1]:
                    cp.start()
        wc[NCH - 1].wait()
        plsc.subcore_barrier()
        @pl.when(sid < 8)
        def _load_e2():
            pltpu.sync_copy(
                e2t_hbm.at[pl.ds(sid, 1)],
                t_sh.at[pl.ds(sid // 4, 1), pl.ds((sid % 4) * 130000, 130000)])
        plsc.subcore_barrier()
        cps = []
        for j in range(NCH):
            cols = pl.ds(j * CH, CH)
            for kk in range(8):
                cps.append(pltpu.make_async_copy(
                    t_sh.at[kk // 4].at[i2k_v.at[j, kk % 4]],
                    g23t_v.at[kk, cols], sem))
        for cp in cps:
            cp.start()
        for cp in cps:
            cp.wait()
        plsc.subcore_barrier()
        @pl.when(sid < 4)
        def _load_e3():
            half = pl.ds((sid % 2) * 405000, 405000)
            pltpu.sync_copy(e3t_hbm.at[pl.ds(sid // 2, 1), half],
                            t_sh.at[pl.ds(sid // 2, 1), half])
        plsc.subcore_barrier()
        cps = []
        for j in range(NCH):
            cols = pl.ds(j * CH, CH)
            for kk in range(2):
                cps.append(pltpu.make_async_copy(
                    t_sh.at[kk].at[i3_v.at[j]], g23t_v.at[8 + kk, cols], sem))
        for cp in cps:
            cp.start()
        for cp in cps:
            cp.wait()
        pltpu.sync_copy(g23t_v, g23t_hbm.at[:, pl.ds(base, TPW)])

    return k(tok, e1t, e2t, e3t)


BM = 2048


def _tc_body(tokr_r, gh_r, g1t_r, g23t_r, wh_r, w1_r, w23_r, out_r):
    tr = tokr_r[...][0:1, :]
    m0r = (tr < 10000).astype(jnp.float32)
    m0full = lax.dot_general(m0r, jnp.ones((1, 128), jnp.float32),
                             (((0,), (0,)), ((), ())),
                             preferred_element_type=jnp.float32)
    acc = jnp.dot(gh_r[...], wh_r[...], preferred_element_type=jnp.float32) * m0full
    m1 = (tr >= 10000) & (tr < 60000)
    g1t = jnp.where(m1, g1t_r[...], 0.0)
    acc += lax.dot_general(g1t, w1_r[...], (((0,), (0,)), ((), ())),
                           preferred_element_type=jnp.float32)
    m2 = (tr >= 60000) & (tr < 190000)
    m3 = tr >= 190000
    row = lax.broadcasted_iota(jnp.int32, (16, BM), 0)
    r8 = row < 8
    m23 = (r8 & m2) | (~r8 & (row < 10) & m3)
    g23t = jnp.where(m23, g23t_r[...], 0.0)
    acc += lax.dot_general(g23t, w23_r[...], (((0,), (0,)), ((), ())),
                           preferred_element_type=jnp.float32)
    out_r[...] = acc


def _tc_project(tokrow, gh, g1t, g23t, head_W, W1, W23):
    grid = (N // BM,)
    return pl.pallas_call(
        _tc_body,
        grid=grid,
        in_specs=[
            pl.BlockSpec((8, BM), lambda i: (0, i)),
            pl.BlockSpec((BM, 128), lambda i: (i, 0)),
            pl.BlockSpec((32, BM), lambda i: (0, i)),
            pl.BlockSpec((16, BM), lambda i: (0, i)),
            pl.BlockSpec((128, 128), lambda i: (0, 0)),
            pl.BlockSpec((32, 128), lambda i: (0, 0)),
            pl.BlockSpec((16, 128), lambda i: (0, 0)),
        ],
        out_specs=pl.BlockSpec((BM, 128), lambda i: (i, 0)),
        out_shape=jax.ShapeDtypeStruct((N, F), jnp.float32),
    )(tokrow, gh, g1t, g23t, head_W, W1, W23)


def kernel(input, head_emb, head_W, emb1, W1, emb2, W2, emb3, W3):
    gh = _sc_head(input, head_emb)
    e1t, e2t, e3t, gh = lax.optimization_barrier((emb1.T, emb2.T, emb3.T, gh))
    g1t, g23t = _sc_tails(input, e1t, e2t, e3t)
    W23 = jnp.concatenate([W2, W3, jnp.zeros((6, 128), jnp.float32)], axis=0)
    tokrow = jnp.broadcast_to(input[None, :], (8, N))
    return _tc_project(tokrow, gh, g1t, g23t, head_W, W1, W23)


# speedup vs baseline: 5.6445x; 1.0510x over previous
import functools

import jax
import jax.numpy as jnp
from jax import lax
from jax.experimental import pallas as pl
from jax.experimental.pallas import tpu as pltpu
from jax.experimental.pallas import tpu_sc as plsc

N = 16384
F = 128
NC, NS = 2, 16
NW = NC * NS
TPW = N // NW
CH = 128
NCH = TPW // CH


def _sc_head(tok, head_emb):
    mesh = plsc.VectorSubcoreMesh(
        core_axis_name="c", subcore_axis_name="s", num_cores=NC, num_subcores=NS
    )

    @functools.partial(
        pl.kernel,
        compiler_params=pltpu.CompilerParams(use_tc_tiling_on_sc=False),
        out_type=jax.ShapeDtypeStruct((N, 128), jnp.float32),
        mesh=mesh,
        scratch_types=[
            pltpu.VMEM((TPW,), jnp.int32),
            pltpu.VMEM((NCH, CH), jnp.int32),
            pltpu.VMEM((2, CH, 128), jnp.float32),
            pltpu.VMEM_SHARED((10000, 128), jnp.float32),
            pltpu.SemaphoreType.DMA,
            pltpu.SemaphoreType.DMA,
            pltpu.SemaphoreType.DMA,
            pltpu.SemaphoreType.DMA,
        ],
    )
    def k(tok_hbm, he_hbm, gh_hbm, tok_v, ih_v, gh_v, he_sh,
          semh0, semh1, semw0, semw1):
        sid = lax.axis_index("s")
        wid = sid * NC + lax.axis_index("c")
        base = wid * TPW
        pltpu.sync_copy(he_hbm.at[pl.ds(sid * 625, 625)],
                        he_sh.at[pl.ds(sid * 625, 625)])
        pltpu.sync_copy(tok_hbm.at[pl.ds(base, TPW)], tok_v)
        for j in range(TPW // 16):
            v = tok_v[pl.ds(j * 16, 16)]
            r, c = divmod(j * 16, CH)
            ih_v[r, pl.ds(c, 16)] = jnp.clip(v, 0, 9999)
        plsc.subcore_barrier()
        semh = [semh0, semh1]
        semw = [semw0, semw1]
        gc = [pltpu.make_async_copy(he_sh.at[ih_v.at[j]], gh_v.at[j % 2],
                                    semh[j % 2]) for j in range(NCH)]
        wc = [pltpu.make_async_copy(gh_v.at[j % 2],
                                    gh_hbm.at[pl.ds(base + j * CH, CH)],
                                    semw[j % 2]) for j in range(NCH)]
        gc[0].start()
        for j in range(NCH):
            gc[j].wait()
            wc[j].start()
            if j + 1 < NCH:
                if j >= 1:
                    wc[j - 1].wait()
                gc[j + 1].start()
        wc[NCH - 1].wait()

    return k(tok, head_emb)


def _sc_tails(tok, e1t, e2t, e3t):
    mesh = plsc.VectorSubcoreMesh(
        core_axis_name="c", subcore_axis_name="s", num_cores=NC, num_subcores=NS
    )

    @functools.partial(
        pl.kernel,
        compiler_params=pltpu.CompilerParams(use_tc_tiling_on_sc=False),
        out_type=(
            jax.ShapeDtypeStruct((32, N), jnp.float32),
            jax.ShapeDtypeStruct((16, N), jnp.float32),
        ),
        mesh=mesh,
        scratch_types=[
            pltpu.VMEM((TPW,), jnp.int32),
            pltpu.VMEM((NCH, 16, CH), jnp.int32),
            pltpu.VMEM((NCH, 4, CH), jnp.int32),
            pltpu.VMEM((NCH, CH), jnp.int32),
            pltpu.VMEM((2, 32, CH), jnp.float32),
            pltpu.VMEM((16, TPW), jnp.float32),
            pltpu.VMEM_SHARED((2, 810000), jnp.float32),
            pltpu.SemaphoreType.DMA,
            pltpu.SemaphoreType.DMA,
            pltpu.SemaphoreType.DMA,
            pltpu.SemaphoreType.DMA,
            pltpu.SemaphoreType.DMA,
        ],
    )
    def k(tok_hbm, e1t_hbm, e2t_hbm, e3t_hbm, g1t_hbm, g23t_hbm,
          tok_v, i1k_v, i2k_v, i3_v, g1t_v, g23t_v, t_sh,
          sem, semst, semg1, semw0, semw1):
        sid = lax.axis_index("s")
        wid = sid * NC + lax.axis_index("c")
        base = wid * TPW
        st1 = pltpu.make_async_copy(
            e1t_hbm.at[pl.ds(sid, 1)],
            t_sh.at[pl.ds(0, 1), pl.ds(sid * 50000, 50000)], semst)
        st2 = pltpu.make_async_copy(
            e1t_hbm.at[pl.ds(sid + 16, 1)],
            t_sh.at[pl.ds(1, 1), pl.ds(sid * 50000, 50000)], semst)
        st1.start()
        st2.start()
        pltpu.sync_copy(tok_hbm.at[pl.ds(base, TPW)], tok_v)
        zeros = jnp.zeros((16,), jnp.float32)
        for j in range(TPW // 16):
            v = tok_v[pl.ds(j * 16, 16)]
            r, c = divmod(j * 16, CH)
            s = pl.ds(c, 16)
            v1 = jnp.clip(v - 10000, 0, 49999)
            for m in range(16):
                i1k_v[r, m, s] = v1 + (m * 50000)
            v2 = jnp.clip(v - 60000, 0, 129999)
            for m in range(4):
                i2k_v[r, m, s] = v2 + (m * 130000)
            i3_v[r, s] = jnp.clip(v - 190000, 0, 809999)
            for z in range(10, 16):
                g23t_v[z, pl.ds(j * 16, 16)] = zeros
        st1.wait()
        st2.wait()
        plsc.subcore_barrier()
        semg = [sem, semg1]
        semw = [semw0, semw1]
        gc = [[pltpu.make_async_copy(
            t_sh.at[kk // 16].at[i1k_v.at[j, kk % 16]],
            g1t_v.at[j % 2, kk], semg[j % 2]) for kk in range(32)]
            for j in range(NCH)]
        wc = [pltpu.make_async_copy(
            g1t_v.at[j % 2], g1t_hbm.at[:, pl.ds(base + j * CH, CH)],
            semw[j % 2]) for j in range(NCH)]
        for cp in gc[0]:
            cp.start()
        for j in range(NCH):
            for cp in gc[j]:
                cp.wait()
            wc[j].start()
            if j + 1 < NCH:
                if j >= 1:
                    wc[j - 1].wait()
                for cp in gc[j + 1]:
                    cp.start()
        wc[NCH - 1].wait()
        plsc.subcore_barrier()
        @pl.when(sid < 8)
        def _load_e2():
            pltpu.sync_copy(
                e2t_hbm.at[pl.ds(sid, 1)],
                t_sh.at[pl.ds(sid // 4, 1), pl.ds((sid % 4) * 130000, 130000)])
        plsc.subcore_barrier()
        cps = []
        for j in range(NCH):
            cols = pl.ds(j * CH, CH)
            for kk in range(8):
                cps.append(pltpu.make_async_copy(
                    t_sh.at[kk // 4].at[i2k_v.at[j, kk % 4]],
                    g23t_v.at[kk, cols], sem))
        for cp in cps:
            cp.start()
        for cp in cps:
            cp.wait()
        plsc.subcore_barrier()
        @pl.when(sid < 4)
        def _load_e3():
            half = pl.ds((sid % 2) * 405000, 405000)
            pltpu.sync_copy(e3t_hbm.at[pl.ds(sid // 2, 1), half],
                            t_sh.at[pl.ds(sid // 2, 1), half])
        plsc.subcore_barrier()
        cps = []
        for j in range(NCH):
            cols = pl.ds(j * CH, CH)
            for kk in range(2):
                cps.append(pltpu.make_async_copy(
                    t_sh.at[kk].at[i3_v.at[j]], g23t_v.at[8 + kk, cols], sem))
        for cp in cps:
            cp.start()
        for cp in cps:
            cp.wait()
        pltpu.sync_copy(g23t_v, g23t_hbm.at[:, pl.ds(base, TPW)])

    return k(tok, e1t, e2t, e3t)


BM = 2048


def _tc_body(tokr_r, gh_r, g1t_r, g23t_r, wh_r, w1_r, w23_r, out_r):
    tr = tokr_r[...][0:1, :]
    m0r = (tr < 10000).astype(jnp.float32)
    m0full = lax.dot_general(m0r, jnp.ones((1, 128), jnp.float32),
                             (((0,), (0,)), ((), ())),
                             preferred_element_type=jnp.float32)
    acc = jnp.dot(gh_r[...], wh_r[...], preferred_element_type=jnp.float32) * m0full
    m1 = (tr >= 10000) & (tr < 60000)
    g1t = jnp.where(m1, g1t_r[...], 0.0)
    acc += lax.dot_general(g1t, w1_r[...], (((0,), (0,)), ((), ())),
                           preferred_element_type=jnp.float32)
    m2 = (tr >= 60000) & (tr < 190000)
    m3 = tr >= 190000
    row = lax.broadcasted_iota(jnp.int32, (16, BM), 0)
    r8 = row < 8
    m23 = (r8 & m2) | (~r8 & (row < 10) & m3)
    g23t = jnp.where(m23, g23t_r[...], 0.0)
    acc += lax.dot_general(g23t, w23_r[...], (((0,), (0,)), ((), ())),
                           preferred_element_type=jnp.float32)
    out_r[...] = acc


def _tc_project(tokrow, gh, g1t, g23t, head_W, W1, W23):
    grid = (N // BM,)
    return pl.pallas_call(
        _tc_body,
        grid=grid,
        in_specs=[
            pl.BlockSpec((8, BM), lambda i: (0, i)),
            pl.BlockSpec((BM, 128), lambda i: (i, 0)),
            pl.BlockSpec((32, BM), lambda i: (0, i)),
            pl.BlockSpec((16, BM), lambda i: (0, i)),
            pl.BlockSpec((128, 128), lambda i: (0, 0)),
            pl.BlockSpec((32, 128), lambda i: (0, 0)),
            pl.BlockSpec((16, 128), lambda i: (0, 0)),
        ],
        out_specs=pl.BlockSpec((BM, 128), lambda i: (i, 0)),
        out_shape=jax.ShapeDtypeStruct((N, F), jnp.float32),
    )(tokrow, gh, g1t, g23t, head_W, W1, W23)


def kernel(input, head_emb, head_W, emb1, W1, emb2, W2, emb3, W3):
    gh = _sc_head(input, head_emb)
    g1t, g23t = _sc_tails(input, emb1.T, emb2.T, emb3.T)
    W23 = jnp.concatenate([W2, W3, jnp.zeros((6, 128), jnp.float32)], axis=0)
    tokrow = jnp.broadcast_to(input[None, :], (8, N))
    return _tc_project(tokrow, gh, g1t, g23t, head_W, W1, W23)


# speedup vs baseline: 5.7314x; 1.0154x over previous
import functools

import jax
import jax.numpy as jnp
from jax import lax
from jax.experimental import pallas as pl
from jax.experimental.pallas import tpu as pltpu
from jax.experimental.pallas import tpu_sc as plsc

N = 16384
F = 128
NC, NS = 2, 16
NW = NC * NS
TPW = N // NW
CH = 128
NCH = TPW // CH


def _sc_head(tok, head_emb):
    mesh = plsc.VectorSubcoreMesh(
        core_axis_name="c", subcore_axis_name="s", num_cores=NC, num_subcores=NS
    )

    @functools.partial(
        pl.kernel,
        compiler_params=pltpu.CompilerParams(use_tc_tiling_on_sc=False),
        out_type=jax.ShapeDtypeStruct((N, 128), jnp.float32),
        mesh=mesh,
        scratch_types=[
            pltpu.VMEM((TPW,), jnp.int32),
            pltpu.VMEM((NCH, CH), jnp.int32),
            pltpu.VMEM((2, CH, 128), jnp.float32),
            pltpu.VMEM_SHARED((10000, 128), jnp.float32),
            pltpu.SemaphoreType.DMA,
            pltpu.SemaphoreType.DMA,
            pltpu.SemaphoreType.DMA,
            pltpu.SemaphoreType.DMA,
        ],
    )
    def k(tok_hbm, he_hbm, gh_hbm, tok_v, ih_v, gh_v, he_sh,
          semh0, semh1, semw0, semw1):
        sid = lax.axis_index("s")
        wid = sid * NC + lax.axis_index("c")
        base = wid * TPW
        pltpu.sync_copy(he_hbm.at[pl.ds(sid * 625, 625)],
                        he_sh.at[pl.ds(sid * 625, 625)])
        pltpu.sync_copy(tok_hbm.at[pl.ds(base, TPW)], tok_v)
        for j in range(TPW // 16):
            v = tok_v[pl.ds(j * 16, 16)]
            r, c = divmod(j * 16, CH)
            ih_v[r, pl.ds(c, 16)] = jnp.clip(v, 0, 9999)
        plsc.subcore_barrier()
        semh = [semh0, semh1]
        semw = [semw0, semw1]
        gc = [pltpu.make_async_copy(he_sh.at[ih_v.at[j]], gh_v.at[j % 2],
                                    semh[j % 2]) for j in range(NCH)]
        wc = [pltpu.make_async_copy(gh_v.at[j % 2],
                                    gh_hbm.at[pl.ds(base + j * CH, CH)],
                                    semw[j % 2]) for j in range(NCH)]
        gc[0].start()
        for j in range(NCH):
            gc[j].wait()
            wc[j].start()
            if j + 1 < NCH:
                if j >= 1:
                    wc[j - 1].wait()
                gc[j + 1].start()
        wc[NCH - 1].wait()

    return k(tok, head_emb)


def _sc_tails(tok, e1t, e2t, e3t):
    mesh = plsc.VectorSubcoreMesh(
        core_axis_name="c", subcore_axis_name="s", num_cores=NC, num_subcores=NS
    )

    @functools.partial(
        pl.kernel,
        compiler_params=pltpu.CompilerParams(use_tc_tiling_on_sc=False),
        out_type=(
            jax.ShapeDtypeStruct((32, N), jnp.float32),
            jax.ShapeDtypeStruct((16, N), jnp.float32),
        ),
        mesh=mesh,
        scratch_types=[
            pltpu.VMEM((TPW,), jnp.int32),
            pltpu.VMEM((NCH, 16, CH), jnp.int32),
            pltpu.VMEM((NCH, 4, CH), jnp.int32),
            pltpu.VMEM((NCH, CH), jnp.int32),
            pltpu.VMEM((2, 32, CH), jnp.float32),
            pltpu.VMEM((16, TPW), jnp.float32),
            pltpu.VMEM_SHARED((2, 810000), jnp.float32),
            pltpu.SemaphoreType.DMA,
            pltpu.SemaphoreType.DMA,
            pltpu.SemaphoreType.DMA,
            pltpu.SemaphoreType.DMA,
            pltpu.SemaphoreType.DMA,
        ],
    )
    def k(tok_hbm, e1t_hbm, e2t_hbm, e3t_hbm, g1t_hbm, g23t_hbm,
          tok_v, i1k_v, i2k_v, i3_v, g1t_v, g23t_v, t_sh,
          sem, semst, semg1, semw0, semw1):
        sid = lax.axis_index("s")
        wid = sid * NC + lax.axis_index("c")
        base = wid * TPW
        st1 = pltpu.make_async_copy(
            e1t_hbm.at[pl.ds(sid, 1)],
            t_sh.at[pl.ds(0, 1), pl.ds(sid * 50000, 50000)], semst)
        st2 = pltpu.make_async_copy(
            e1t_hbm.at[pl.ds(sid + 16, 1)],
            t_sh.at[pl.ds(1, 1), pl.ds(sid * 50000, 50000)], semst)
        st1.start()
        st2.start()
        pltpu.sync_copy(tok_hbm.at[pl.ds(base, TPW)], tok_v)
        zeros = jnp.zeros((16,), jnp.float32)
        for j in range(TPW // 16):
            v = tok_v[pl.ds(j * 16, 16)]
            r, c = divmod(j * 16, CH)
            s = pl.ds(c, 16)
            v1 = jnp.clip(v - 10000, 0, 49999)
            for m in range(16):
                i1k_v[r, m, s] = v1 + (m * 50000)
            v2 = jnp.clip(v - 60000, 0, 129999)
            for m in range(4):
                i2k_v[r, m, s] = v2 + (m * 130000)
            i3_v[r, s] = jnp.clip(v - 190000, 0, 809999)
            for z in range(10, 16):
                g23t_v[z, pl.ds(j * 16, 16)] = zeros
        st1.wait()
        st2.wait()
        plsc.subcore_barrier()
        semg = [sem, semg1]
        semw = [semw0, semw1]
        gc = [[pltpu.make_async_copy(
            t_sh.at[kk // 16].at[i1k_v.at[j, kk % 16]],
            g1t_v.at[j % 2, kk], semg[j % 2]) for kk in range(32)]
            for j in range(NCH)]
        wc = [pltpu.make_async_copy(
            g1t_v.at[j % 2], g1t_hbm.at[:, pl.ds(base + j * CH, CH)],
            semw[j % 2]) for j in range(NCH)]
        for cp in gc[0]:
            cp.start()
        for j in range(NCH):
            for cp in gc[j]:
                cp.wait()
            wc[j].start()
            if j + 1 < NCH:
                if j >= 1:
                    wc[j - 1].wait()
                for cp in gc[j + 1]:
                    cp.start()
        wc[NCH - 1].wait()
        plsc.subcore_barrier()
        @pl.when(sid < 8)
        def _load_e2():
            pltpu.sync_copy(
                e2t_hbm.at[pl.ds(sid, 1)],
                t_sh.at[pl.ds(sid // 4, 1), pl.ds((sid % 4) * 130000, 130000)])
        plsc.subcore_barrier()
        cps = []
        for j in range(NCH):
            cols = pl.ds(j * CH, CH)
            for kk in range(8):
                cps.append(pltpu.make_async_copy(
                    t_sh.at[kk // 4].at[i2k_v.at[j, kk % 4]],
                    g23t_v.at[kk, cols], sem))
        for cp in cps:
            cp.start()
        for cp in cps:
            cp.wait()
        plsc.subcore_barrier()
        @pl.when(sid < 4)
        def _load_e3():
            half = pl.ds((sid % 2) * 405000, 405000)
            pltpu.sync_copy(e3t_hbm.at[pl.ds(sid // 2, 1), half],
                            t_sh.at[pl.ds(sid // 2, 1), half])
        plsc.subcore_barrier()
        cps = []
        for j in range(NCH):
            cols = pl.ds(j * CH, CH)
            for kk in range(2):
                cps.append(pltpu.make_async_copy(
                    t_sh.at[kk].at[i3_v.at[j]], g23t_v.at[8 + kk, cols], sem))
        for cp in cps:
            cp.start()
        for cp in cps:
            cp.wait()
        pltpu.sync_copy(g23t_v, g23t_hbm.at[:, pl.ds(base, TPW)])

    return k(tok, e1t, e2t, e3t)


BM = 4096


def _tc_body(tokr_r, gh_r, g1t_r, g23t_r, wh_r, w1_r, w23_r, out_r):
    tr = tokr_r[...][0:1, :]
    m0r = (tr < 10000).astype(jnp.float32)
    m0full = lax.dot_general(m0r, jnp.ones((1, 128), jnp.float32),
                             (((0,), (0,)), ((), ())),
                             preferred_element_type=jnp.float32)
    acc = jnp.dot(gh_r[...], wh_r[...], preferred_element_type=jnp.float32) * m0full
    m1 = (tr >= 10000) & (tr < 60000)
    g1t = jnp.where(m1, g1t_r[...], 0.0)
    acc += lax.dot_general(g1t, w1_r[...], (((0,), (0,)), ((), ())),
                           preferred_element_type=jnp.float32)
    m2 = (tr >= 60000) & (tr < 190000)
    m3 = tr >= 190000
    row = lax.broadcasted_iota(jnp.int32, (16, BM), 0)
    r8 = row < 8
    m23 = (r8 & m2) | (~r8 & (row < 10) & m3)
    g23t = jnp.where(m23, g23t_r[...], 0.0)
    acc += lax.dot_general(g23t, w23_r[...], (((0,), (0,)), ((), ())),
                           preferred_element_type=jnp.float32)
    out_r[...] = acc


def _tc_project(tokrow, gh, g1t, g23t, head_W, W1, W23):
    grid = (N // BM,)
    return pl.pallas_call(
        _tc_body,
        grid=grid,
        in_specs=[
            pl.BlockSpec((8, BM), lambda i: (0, i)),
            pl.BlockSpec((BM, 128), lambda i: (i, 0)),
            pl.BlockSpec((32, BM), lambda i: (0, i)),
            pl.BlockSpec((16, BM), lambda i: (0, i)),
            pl.BlockSpec((128, 128), lambda i: (0, 0)),
            pl.BlockSpec((32, 128), lambda i: (0, 0)),
            pl.BlockSpec((16, 128), lambda i: (0, 0)),
        ],
        out_specs=pl.BlockSpec((BM, 128), lambda i: (i, 0)),
        out_shape=jax.ShapeDtypeStruct((N, F), jnp.float32),
    )(tokrow, gh, g1t, g23t, head_W, W1, W23)


def kernel(input, head_emb, head_W, emb1, W1, emb2, W2, emb3, W3):
    gh = _sc_head(input, head_emb)
    g1t, g23t = _sc_tails(input, emb1.T, emb2.T, emb3.T)
    W23 = jnp.concatenate([W2, W3, jnp.zeros((6, 128), jnp.float32)], axis=0)
    tokrow = jnp.broadcast_to(input[None, :], (8, N))
    return _tc_project(tokrow, gh, g1t, g23t, head_W, W1, W23)
